# baseline (speedup 1.0000x reference)
import jax
import jax.numpy as jnp
from jax.experimental import pallas as pl
from jax.experimental.pallas import tpu as pltpu

VOCAB = 100000
D = 64
NCTX = 20
HID = 128
VB = 8192
NVB = VOCAB // VB
TAIL0 = NVB * VB
TAIL = VOCAB - TAIL0
TB = 2048


def _cbow_kernel(idx_ref, emb_hbm, w1_ref, b1_ref, w2_hbm, w2tail_ref,
                 b2_ref, out_ref, rows_ref, wbuf_ref, row_sem, w2_sem):
    row_copies = [
        pltpu.make_async_copy(
            emb_hbm.at[pl.ds(idx_ref[k], 1), :],
            rows_ref.at[pl.ds(k, 1), :],
            row_sem,
        )
        for k in range(NCTX)
    ]
    for c in row_copies:
        c.start()

    w2_copies = [
        pltpu.make_async_copy(
            w2_hbm.at[:, pl.ds(i * VB, VB)],
            wbuf_ref.at[i],
            w2_sem.at[i],
        )
        for i in range(NVB)
    ]
    for c in w2_copies:
        c.start()

    for c in row_copies:
        c.wait()
    h = b1_ref[...]
    for k in range(NCTX):
        h = h + jnp.dot(rows_ref[pl.ds(k, 1), :],
                        w1_ref[pl.ds(k * D, D), :],
                        preferred_element_type=jnp.float32)
    h = jnp.maximum(h, 0.0)

    zt = jnp.dot(h, w2tail_ref[...], preferred_element_type=jnp.float32)
    zt = zt[:, :TAIL] + b2_ref[:, TAIL0:]
    m = jnp.max(zt)
    s = jnp.sum(jnp.exp(zt - m))
    out_ref[:, TAIL0:] = zt

    for i in range(NVB):
        w2_copies[i].wait()
        z = jnp.dot(h, wbuf_ref[i], preferred_element_type=jnp.float32)
        z = z + b2_ref[:, i * VB:(i + 1) * VB]
        m_new = jnp.maximum(m, jnp.max(z))
        s = s * jnp.exp(m - m_new) + jnp.sum(jnp.exp(z - m_new))
        m = m_new
        out_ref[:, i * VB:(i + 1) * VB] = z

    out_ref[...] = out_ref[...] - (m + jnp.log(s))


def kernel(inputs, emb_table, W1, b1, W2, b2):
    idx = inputs.astype(jnp.int32)

    return pl.pallas_call(
        _cbow_kernel,
        grid=(1,),
        in_specs=[
            pl.BlockSpec(memory_space=pltpu.SMEM),
            pl.BlockSpec(memory_space=pltpu.MemorySpace.HBM),
            pl.BlockSpec(memory_space=pltpu.VMEM),
            pl.BlockSpec(memory_space=pltpu.VMEM),
            pl.BlockSpec(memory_space=pltpu.MemorySpace.HBM),
            pl.BlockSpec((HID, TB), lambda g: (0, TAIL0 // TB)),
            pl.BlockSpec(memory_space=pltpu.VMEM),
        ],
        out_specs=pl.BlockSpec(memory_space=pltpu.VMEM),
        out_shape=jax.ShapeDtypeStruct((1, VOCAB), jnp.float32),
        scratch_shapes=[
            pltpu.VMEM((NCTX, D), jnp.float32),
            pltpu.VMEM((NVB, HID, VB), jnp.float32),
            pltpu.SemaphoreType.DMA,
            pltpu.SemaphoreType.DMA((NVB,)),
        ],
    )(idx, emb_table, W1, b1.reshape(1, HID), W2, W2, b2.reshape(1, VOCAB))
